# baseline (speedup 1.0000x reference)
import functools

import jax
import jax.numpy as jnp
from jax import lax
from jax.experimental import pallas as pl
from jax.experimental.pallas import tpu as pltpu
from jax.experimental.pallas import tpu_sc as plsc

N = 10000
NP = 10240
E = 320000
NC = 2
NS = 16
NW = NC * NS

K = 128
EPT = E // NS
NCH = 160
EPTP = NCH * K
NCHB = 80
EPW = E // NW
NCH4 = 80
EPWP = NCH4 * K
RPT = NP // NS
RB = 2048
GRID = NP // RB
RB5 = 2000

_f32 = jnp.float32


def _mesh():
    return plsc.VectorSubcoreMesh(
        core_axis_name="c", subcore_axis_name="s", num_cores=NC, num_subcores=NS
    )


@functools.partial(
    pl.kernel,
    out_type=jax.ShapeDtypeStruct((NW, NP), _f32),
    mesh=_mesh(),
    scratch_types=[
        pltpu.VMEM((EPWP,), jnp.int32),
        pltpu.VMEM((NP,), _f32),
    ],
    compiler_params=pltpu.CompilerParams(needs_layout_passes=False),
)
def _deg_kernel(dst_hbm, out_hbm, idx_v, deg_v):
    c = lax.axis_index("c")
    s = lax.axis_index("s")
    wid = c * NS + s
    pltpu.sync_copy(dst_hbm.at[wid], idx_v)

    def zero(i, _):
        deg_v[pl.ds(i * 16, 16)] = jnp.zeros((16,), _f32)
        return 0

    lax.fori_loop(0, NP // 16, zero, 0)
    ones = jnp.ones((16,), _f32)

    def body(i, _):
        ids = idx_v[pl.ds(i * 16, 16)]
        plsc.addupdate_scatter(deg_v, [ids], ones)
        return 0

    lax.fori_loop(0, EPWP // 16, body, 0)
    pltpu.sync_copy(deg_v, out_hbm.at[wid])


def _make_scatter(dh, nb, nchb, nch_total, edgesplit):
    npass = nch_total // nchb
    G = nchb // nb

    @functools.partial(
        pl.kernel,
        out_type=jax.ShapeDtypeStruct((NC, NP, dh), _f32),
        mesh=_mesh(),
        scratch_types=[
            pltpu.VMEM((nchb, K), jnp.int32),
            pltpu.VMEM((nchb, K), jnp.int32),
        ]
        + [pltpu.VMEM((128, dh), _f32) for _ in range(nb)]
        + [pltpu.VMEM_SHARED((NP, dh), _f32)]
        + [pltpu.SemaphoreType.DMA for _ in range(2 * nb)],
        compiler_params=pltpu.CompilerParams(use_tc_tiling_on_sc=False),
    )
    def k(g_hbm, src_hbm, dst_hbm, out_hbm, src_v, dst_v, *rest):
        rawbufs = rest[:nb]
        acc = rest[nb]
        gsems = rest[nb + 1 : 2 * nb + 1]
        ssems = rest[2 * nb + 1 :]
        c = lax.axis_index("c")
        s = lax.axis_index("s")
        bufs = rawbufs
        buf0 = rawbufs[0]
        if edgesplit:
            gtab = g_hbm
            src_idx = src_hbm.at[c].at[s]
            dst_idx = dst_hbm.at[c].at[s]
        else:
            gtab = g_hbm.at[c]
            src_idx = src_hbm.at[s]
            dst_idx = dst_hbm.at[s]

        def issue_g(j, b):
            pltpu.async_copy(gtab.at[src_v.at[j]], bufs[b], gsems[b])

        def wait_g(j, b):
            pltpu.make_async_copy(gtab.at[src_v.at[j]], bufs[b], gsems[b]).wait()

        def issue_s(j, b):
            pltpu.async_copy(bufs[b], acc.at[dst_v.at[j]], ssems[b], add=True)

        def wait_s(j, b):
            pltpu.make_async_copy(bufs[b], acc.at[dst_v.at[j]], ssems[b]).wait()

        base = s * RPT
        if edgesplit:
            @pl.when(c == 0)
            def _():
                pltpu.sync_copy(gtab.at[pl.ds(base, RPT)], acc.at[pl.ds(base, RPT)])

            @pl.when(c != 0)
            def _():
                def zrow(i, _):
                    for j in range(dh // 16):
                        buf0[i, pl.ds(j * 16, 16)] = jnp.zeros((16,), _f32)
                    return 0

                lax.fori_loop(0, 128, zrow, 0)
                for r in range(RPT // 128):
                    pltpu.sync_copy(buf0, acc.at[pl.ds(base + r * 128, 128)])
        else:
            pltpu.sync_copy(gtab.at[pl.ds(base, RPT)], acc.at[pl.ds(base, RPT)])
        plsc.subcore_barrier()

        for p in range(npass):
            pltpu.sync_copy(src_idx.at[pl.ds(p * nchb, nchb)], src_v)
            pltpu.sync_copy(dst_idx.at[pl.ds(p * nchb, nchb)], dst_v)
            for b in range(nb):
                issue_g(b, b)

            def body(gi, _):
                for b in range(nb):
                    j = nb * gi + b
                    wait_g(j, b)
                    issue_s(j, b)
                    wait_s(j, b)
                    issue_g(j + nb, b)
                return 0

            lax.fori_loop(0, G - 1, body, 0)
            for b in range(nb):
                j = nb * (G - 1) + b
                wait_g(j, b)
                issue_s(j, b)
                wait_s(j, b)
        plsc.subcore_barrier()
        pltpu.sync_copy(acc.at[pl.ds(base, RPT)], out_hbm.at[c].at[pl.ds(base, RPT)])

    return k


def _tc1_body(deg_ref, x_ref, wa_ref, wb_ref, g_ref, dinv_ref):
    deg = jnp.sum(deg_ref[...], axis=0)[:, None] + 1.0
    dinv = lax.rsqrt(deg)
    dinv_ref[...] = dinv
    xa = x_ref[...]
    g_ref[0] = dinv * jnp.dot(xa, wa_ref[...], preferred_element_type=_f32)
    g_ref[1] = dinv * jnp.dot(xa, wb_ref[...], preferred_element_type=_f32)


def _tc_mid_body(s_ref, dinv_ref, b_ref, wa_ref, wb_ref, o_ref):
    dinv = dinv_ref[...]
    t = jnp.concatenate([s_ref[0], s_ref[1]], axis=1)
    a = jnp.maximum(dinv * t + b_ref[...], 0.0)
    o_ref[0] = dinv * jnp.dot(a, wa_ref[...], preferred_element_type=_f32)
    o_ref[1] = dinv * jnp.dot(a, wb_ref[...], preferred_element_type=_f32)


def _tc4_body(s_ref, dinv_ref, b_ref, w_ref, o_ref):
    dinv = dinv_ref[...]
    t = jnp.concatenate([s_ref[0], s_ref[1]], axis=1)
    a = jnp.maximum(dinv * t + b_ref[...], 0.0)
    o_ref[...] = dinv * jnp.dot(a, w_ref[...], preferred_element_type=_f32)


def _tc5_body(s_ref, dinv_ref, b_ref, o_ref):
    dinv = dinv_ref[...]
    logits = dinv * (s_ref[0] + s_ref[1]) + b_ref[...]
    m = jnp.max(logits, axis=1, keepdims=True)
    lse = jnp.log(jnp.sum(jnp.exp(logits - m), axis=1, keepdims=True)) + m
    o_ref[...] = logits - lse


def _pad2(w, rows, cols):
    return jnp.pad(w, ((0, rows - w.shape[0]), (0, cols - w.shape[1])))


def kernel(x, edge_index, W1, b1, W2, b2, W3, b3, W4, b4):
    x = jnp.pad(x, ((0, NP - N), (0, 0)))
    src = edge_index[0]
    dst = edge_index[1]
    ghost = NP - 1
    src_t = jnp.pad(src.reshape(NS, EPT), ((0, 0), (0, EPTP - EPT)),
                    constant_values=ghost).reshape(NS, NCH, K)
    dst_t = jnp.pad(dst.reshape(NS, EPT), ((0, 0), (0, EPTP - EPT)),
                    constant_values=ghost).reshape(NS, NCH, K)
    src_w = jnp.pad(src.reshape(NW, EPW), ((0, 0), (0, EPWP - EPW)),
                    constant_values=ghost)
    dst_w = jnp.pad(dst.reshape(NW, EPW), ((0, 0), (0, EPWP - EPW)),
                    constant_values=ghost)
    dst_deg = dst_w
    src_e = src_w.reshape(NC, NS, NCH4, K)
    dst_e = dst_w.reshape(NC, NS, NCH4, K)

    w1 = _pad2(W1, 128, 224)
    w2 = _pad2(W2, 224, 160)
    w3 = _pad2(W3, 160, 64)
    w4 = _pad2(W4, 64, 16)
    b1p = jnp.pad(b1, (0, 4)).reshape(1, 224)
    b2p = jnp.pad(b2, (0, 10)).reshape(1, 160)
    b3p = jnp.pad(b3, (0, 4)).reshape(1, 64)
    b4p = b4.reshape(1, 16)

    deg_parts = _deg_kernel(dst_deg)

    g1, dinv = pl.pallas_call(
        _tc1_body,
        grid=(GRID,),
        in_specs=[
            pl.BlockSpec((NW, RB), lambda i: (0, i)),
            pl.BlockSpec((RB, 128), lambda i: (i, 0)),
            pl.BlockSpec((128, 112), lambda i: (0, 0)),
            pl.BlockSpec((128, 112), lambda i: (0, 0)),
        ],
        out_specs=[
            pl.BlockSpec((2, RB, 112), lambda i: (0, i, 0)),
            pl.BlockSpec((RB, 1), lambda i: (i, 0)),
        ],
        out_shape=[
            jax.ShapeDtypeStruct((2, NP, 112), _f32),
            jax.ShapeDtypeStruct((NP, 1), _f32),
        ],
    )(deg_parts, x, w1[:, :112], w1[:, 112:])

    s1 = _scat112(g1, src_t, dst_t)

    g2 = pl.pallas_call(
        _tc_mid_body,
        grid=(GRID,),
        in_specs=[
            pl.BlockSpec((2, RB, 112), lambda i: (0, i, 0)),
            pl.BlockSpec((RB, 1), lambda i: (i, 0)),
            pl.BlockSpec((1, 224), lambda i: (0, 0)),
            pl.BlockSpec((224, 80), lambda i: (0, 0)),
            pl.BlockSpec((224, 80), lambda i: (0, 0)),
        ],
        out_specs=pl.BlockSpec((2, RB, 80), lambda i: (0, i, 0)),
        out_shape=jax.ShapeDtypeStruct((2, NP, 80), _f32),
    )(s1, dinv, b1p, w2[:, :80], w2[:, 80:])

    s2 = _scat80(g2, src_t, dst_t)

    g3 = pl.pallas_call(
        _tc_mid_body,
        grid=(GRID,),
        in_specs=[
            pl.BlockSpec((2, RB, 80), lambda i: (0, i, 0)),
            pl.BlockSpec((RB, 1), lambda i: (i, 0)),
            pl.BlockSpec((1, 160), lambda i: (0, 0)),
            pl.BlockSpec((160, 32), lambda i: (0, 0)),
            pl.BlockSpec((160, 32), lambda i: (0, 0)),
        ],
        out_specs=pl.BlockSpec((2, RB, 32), lambda i: (0, i, 0)),
        out_shape=jax.ShapeDtypeStruct((2, NP, 32), _f32),
    )(s2, dinv, b2p, w3[:, :32], w3[:, 32:])

    s3 = _scat32(g3, src_t, dst_t)

    g4 = pl.pallas_call(
        _tc4_body,
        grid=(GRID,),
        in_specs=[
            pl.BlockSpec((2, RB, 32), lambda i: (0, i, 0)),
            pl.BlockSpec((RB, 1), lambda i: (i, 0)),
            pl.BlockSpec((1, 64), lambda i: (0, 0)),
            pl.BlockSpec((64, 16), lambda i: (0, 0)),
        ],
        out_specs=pl.BlockSpec((RB, 16), lambda i: (i, 0)),
        out_shape=jax.ShapeDtypeStruct((NP, 16), _f32),
    )(s3, dinv, b3p, w4)

    s4 = _scat16e(g4, src_e, dst_e)

    out = pl.pallas_call(
        _tc5_body,
        grid=(N // RB5,),
        in_specs=[
            pl.BlockSpec((2, RB5, 16), lambda i: (0, i, 0)),
            pl.BlockSpec((RB5, 1), lambda i: (i, 0)),
            pl.BlockSpec((1, 16), lambda i: (0, 0)),
        ],
        out_specs=pl.BlockSpec((RB5, 16), lambda i: (i, 0)),
        out_shape=jax.ShapeDtypeStruct((N, 16), _f32),
    )(s4, dinv, b4p)

    return out


_scat112 = _make_scatter(112, 2, 80, NCH, False)
_scat80 = _make_scatter(80, 4, 80, NCH, False)
_scat32 = _make_scatter(32, 4, 160, NCH, False)
_scat16e = _make_scatter(16, 4, 80, NCH4, True)


# speedup vs baseline: 1.9621x; 1.9621x over previous
import functools

import jax
import jax.numpy as jnp
from jax import lax
from jax.experimental import pallas as pl
from jax.experimental.pallas import tpu as pltpu
from jax.experimental.pallas import tpu_sc as plsc

N = 10000
NP = 10240
E = 320000
NC = 2
NS = 16
NW = NC * NS

K = 125
EPT = E // NS
NCH = EPT // K
NCHB = 80
EPW = E // NW
NCH4 = EPW // K
RPT = NP // NS
RB = 2048
GRID = NP // RB
RB5 = 2000

_f32 = jnp.float32


def _mesh():
    return plsc.VectorSubcoreMesh(
        core_axis_name="c", subcore_axis_name="s", num_cores=NC, num_subcores=NS
    )


@functools.partial(
    pl.kernel,
    out_type=jax.ShapeDtypeStruct((NW, NP), _f32),
    mesh=_mesh(),
    scratch_types=[
        pltpu.VMEM((EPW,), jnp.int32),
        pltpu.VMEM((NP,), _f32),
    ],
    compiler_params=pltpu.CompilerParams(needs_layout_passes=False),
)
def _deg_kernel(dst_hbm, out_hbm, idx_v, deg_v):
    c = lax.axis_index("c")
    s = lax.axis_index("s")
    wid = c * NS + s
    pltpu.sync_copy(dst_hbm.at[wid], idx_v)

    def zero(i, _):
        deg_v[pl.ds(i * 16, 16)] = jnp.zeros((16,), _f32)
        return 0

    lax.fori_loop(0, NP // 16, zero, 0)
    ones = jnp.ones((16,), _f32)

    def body(i, _):
        ids = idx_v[pl.ds(i * 16, 16)]
        plsc.addupdate_scatter(deg_v, [ids], ones)
        return 0

    lax.fori_loop(0, EPW // 16, body, 0)
    pltpu.sync_copy(deg_v, out_hbm.at[wid])


def _make_scatter(dh, nb, nchb, nch_total, edgesplit):
    npass = nch_total // nchb
    G = nchb // nb

    @functools.partial(
        pl.kernel,
        out_type=jax.ShapeDtypeStruct((NC, NP, dh), _f32),
        mesh=_mesh(),
        scratch_types=[
            pltpu.VMEM((nchb, K), jnp.int32),
            pltpu.VMEM((nchb, K), jnp.int32),
        ]
        + [pltpu.VMEM((128, dh), _f32) for _ in range(nb)]
        + [pltpu.VMEM_SHARED((NP, dh), _f32)]
        + [pltpu.SemaphoreType.DMA for _ in range(2 * nb)],
        compiler_params=pltpu.CompilerParams(use_tc_tiling_on_sc=False),
    )
    def k(g_hbm, src_hbm, dst_hbm, out_hbm, src_v, dst_v, *rest):
        rawbufs = rest[:nb]
        acc = rest[nb]
        gsems = rest[nb + 1 : 2 * nb + 1]
        ssems = rest[2 * nb + 1 :]
        c = lax.axis_index("c")
        s = lax.axis_index("s")
        bufs = tuple(rb.at[pl.ds(0, K)] for rb in rawbufs)
        buf0 = rawbufs[0]
        if edgesplit:
            gtab = g_hbm
            src_idx = src_hbm.at[c].at[s]
            dst_idx = dst_hbm.at[c].at[s]
        else:
            gtab = g_hbm.at[c]
            src_idx = src_hbm.at[s]
            dst_idx = dst_hbm.at[s]

        def issue_g(j, b):
            pltpu.async_copy(gtab.at[src_v.at[j]], bufs[b], gsems[b])

        def wait_g(j, b):
            pltpu.make_async_copy(gtab.at[src_v.at[j]], bufs[b], gsems[b]).wait()

        def issue_s(j, b):
            pltpu.async_copy(bufs[b], acc.at[dst_v.at[j]], ssems[b], add=True)

        def wait_s(j, b):
            pltpu.make_async_copy(bufs[b], acc.at[dst_v.at[j]], ssems[b]).wait()

        base = s * RPT
        if edgesplit:
            @pl.when(c == 0)
            def _():
                pltpu.sync_copy(gtab.at[pl.ds(base, RPT)], acc.at[pl.ds(base, RPT)])

            @pl.when(c != 0)
            def _():
                def zrow(i, _):
                    for j in range(dh // 16):
                        buf0[i, pl.ds(j * 16, 16)] = jnp.zeros((16,), _f32)
                    return 0

                lax.fori_loop(0, 128, zrow, 0)
                for r in range(RPT // 128):
                    pltpu.sync_copy(buf0, acc.at[pl.ds(base + r * 128, 128)])
        else:
            pltpu.sync_copy(gtab.at[pl.ds(base, RPT)], acc.at[pl.ds(base, RPT)])
        plsc.subcore_barrier()

        for p in range(npass):
            pltpu.sync_copy(src_idx.at[pl.ds(p * nchb, nchb)], src_v)
            pltpu.sync_copy(dst_idx.at[pl.ds(p * nchb, nchb)], dst_v)
            for b in range(nb):
                issue_g(b, b)

            def body(gi, _):
                for b in range(nb):
                    j = nb * gi + b
                    wait_g(j, b)
                    issue_s(j, b)
                    wait_s(j, b)
                    issue_g(j + nb, b)
                return 0

            lax.fori_loop(0, G - 1, body, 0)
            for b in range(nb):
                j = nb * (G - 1) + b
                wait_g(j, b)
                issue_s(j, b)
                wait_s(j, b)
        plsc.subcore_barrier()
        pltpu.sync_copy(acc.at[pl.ds(base, RPT)], out_hbm.at[c].at[pl.ds(base, RPT)])

    return k


def _tc1_body(deg_ref, x_ref, wa_ref, wb_ref, g_ref, dinv_ref):
    deg = jnp.sum(deg_ref[...], axis=0)[:, None] + 1.0
    dinv = lax.rsqrt(deg)
    dinv_ref[...] = dinv
    xa = x_ref[...]
    g_ref[0] = dinv * jnp.dot(xa, wa_ref[...], preferred_element_type=_f32)
    g_ref[1] = dinv * jnp.dot(xa, wb_ref[...], preferred_element_type=_f32)


def _tc_mid_body(s_ref, dinv_ref, b_ref, wa_ref, wb_ref, o_ref):
    dinv = dinv_ref[...]
    t = jnp.concatenate([s_ref[0], s_ref[1]], axis=1)
    a = jnp.maximum(dinv * t + b_ref[...], 0.0)
    o_ref[0] = dinv * jnp.dot(a, wa_ref[...], preferred_element_type=_f32)
    o_ref[1] = dinv * jnp.dot(a, wb_ref[...], preferred_element_type=_f32)


def _tc4_body(s_ref, dinv_ref, b_ref, w_ref, o_ref):
    dinv = dinv_ref[...]
    t = jnp.concatenate([s_ref[0], s_ref[1]], axis=1)
    a = jnp.maximum(dinv * t + b_ref[...], 0.0)
    o_ref[...] = dinv * jnp.dot(a, w_ref[...], preferred_element_type=_f32)


def _tc5_body(s_ref, dinv_ref, b_ref, o_ref):
    dinv = dinv_ref[...]
    logits = dinv * (s_ref[0] + s_ref[1]) + b_ref[...]
    m = jnp.max(logits, axis=1, keepdims=True)
    lse = jnp.log(jnp.sum(jnp.exp(logits - m), axis=1, keepdims=True)) + m
    o_ref[...] = logits - lse


def _pad2(w, rows, cols):
    return jnp.pad(w, ((0, rows - w.shape[0]), (0, cols - w.shape[1])))


def kernel(x, edge_index, W1, b1, W2, b2, W3, b3, W4, b4):
    x = jnp.pad(x, ((0, NP - N), (0, 0)))
    src = edge_index[0]
    dst = edge_index[1]
    dst_deg = dst.reshape(NW, EPW)
    src_t = src.reshape(NS, NCH, K)
    dst_t = dst.reshape(NS, NCH, K)
    src_e = src.reshape(NC, NS, NCH4, K)
    dst_e = dst.reshape(NC, NS, NCH4, K)

    w1 = _pad2(W1, 128, 224)
    w2 = _pad2(W2, 224, 160)
    w3 = _pad2(W3, 160, 64)
    w4 = _pad2(W4, 64, 16)
    b1p = jnp.pad(b1, (0, 4)).reshape(1, 224)
    b2p = jnp.pad(b2, (0, 10)).reshape(1, 160)
    b3p = jnp.pad(b3, (0, 4)).reshape(1, 64)
    b4p = b4.reshape(1, 16)

    deg_parts = _deg_kernel(dst_deg)

    g1, dinv = pl.pallas_call(
        _tc1_body,
        grid=(GRID,),
        in_specs=[
            pl.BlockSpec((NW, RB), lambda i: (0, i)),
            pl.BlockSpec((RB, 128), lambda i: (i, 0)),
            pl.BlockSpec((128, 112), lambda i: (0, 0)),
            pl.BlockSpec((128, 112), lambda i: (0, 0)),
        ],
        out_specs=[
            pl.BlockSpec((2, RB, 112), lambda i: (0, i, 0)),
            pl.BlockSpec((RB, 1), lambda i: (i, 0)),
        ],
        out_shape=[
            jax.ShapeDtypeStruct((2, NP, 112), _f32),
            jax.ShapeDtypeStruct((NP, 1), _f32),
        ],
    )(deg_parts, x, w1[:, :112], w1[:, 112:])

    s1 = _scat112(g1, src_t, dst_t)

    g2 = pl.pallas_call(
        _tc_mid_body,
        grid=(GRID,),
        in_specs=[
            pl.BlockSpec((2, RB, 112), lambda i: (0, i, 0)),
            pl.BlockSpec((RB, 1), lambda i: (i, 0)),
            pl.BlockSpec((1, 224), lambda i: (0, 0)),
            pl.BlockSpec((224, 80), lambda i: (0, 0)),
            pl.BlockSpec((224, 80), lambda i: (0, 0)),
        ],
        out_specs=pl.BlockSpec((2, RB, 80), lambda i: (0, i, 0)),
        out_shape=jax.ShapeDtypeStruct((2, NP, 80), _f32),
    )(s1, dinv, b1p, w2[:, :80], w2[:, 80:])

    s2 = _scat80(g2, src_t, dst_t)

    g3 = pl.pallas_call(
        _tc_mid_body,
        grid=(GRID,),
        in_specs=[
            pl.BlockSpec((2, RB, 80), lambda i: (0, i, 0)),
            pl.BlockSpec((RB, 1), lambda i: (i, 0)),
            pl.BlockSpec((1, 160), lambda i: (0, 0)),
            pl.BlockSpec((160, 32), lambda i: (0, 0)),
            pl.BlockSpec((160, 32), lambda i: (0, 0)),
        ],
        out_specs=pl.BlockSpec((2, RB, 32), lambda i: (0, i, 0)),
        out_shape=jax.ShapeDtypeStruct((2, NP, 32), _f32),
    )(s2, dinv, b2p, w3[:, :32], w3[:, 32:])

    s3 = _scat32(g3, src_t, dst_t)

    g4 = pl.pallas_call(
        _tc4_body,
        grid=(GRID,),
        in_specs=[
            pl.BlockSpec((2, RB, 32), lambda i: (0, i, 0)),
            pl.BlockSpec((RB, 1), lambda i: (i, 0)),
            pl.BlockSpec((1, 64), lambda i: (0, 0)),
            pl.BlockSpec((64, 16), lambda i: (0, 0)),
        ],
        out_specs=pl.BlockSpec((RB, 16), lambda i: (i, 0)),
        out_shape=jax.ShapeDtypeStruct((NP, 16), _f32),
    )(s3, dinv, b3p, w4)

    s4 = _scat16e(g4, src_e, dst_e)

    out = pl.pallas_call(
        _tc5_body,
        grid=(N // RB5,),
        in_specs=[
            pl.BlockSpec((2, RB5, 16), lambda i: (0, i, 0)),
            pl.BlockSpec((RB5, 1), lambda i: (i, 0)),
            pl.BlockSpec((1, 16), lambda i: (0, 0)),
        ],
        out_specs=pl.BlockSpec((RB5, 16), lambda i: (i, 0)),
        out_shape=jax.ShapeDtypeStruct((N, 16), _f32),
    )(s4, dinv, b4p)

    return out


_scat112 = _make_scatter(112, 2, 80, NCH, False)
_scat80 = _make_scatter(80, 4, 80, NCH, False)
_scat32 = _make_scatter(32, 4, 160, NCH, False)
_scat16e = _make_scatter(16, 4, 80, NCH4, True)


# speedup vs baseline: 1.9647x; 1.0013x over previous
import functools

import jax
import jax.numpy as jnp
from jax import lax
from jax.experimental import pallas as pl
from jax.experimental.pallas import tpu as pltpu
from jax.experimental.pallas import tpu_sc as plsc

N = 10000
NP = 10240
E = 320000
NC = 2
NS = 16
NW = NC * NS

K = 125
EPT = E // NS
NCH = EPT // K
NCHB = 80
EPW = E // NW
NCH4 = EPW // K
RPT = NP // NS
RB = 2048
GRID = NP // RB
RB5 = 2000

_f32 = jnp.float32


def _mesh():
    return plsc.VectorSubcoreMesh(
        core_axis_name="c", subcore_axis_name="s", num_cores=NC, num_subcores=NS
    )


@functools.partial(
    pl.kernel,
    out_type=jax.ShapeDtypeStruct((NW, NP), _f32),
    mesh=_mesh(),
    scratch_types=[
        pltpu.VMEM((EPW,), jnp.int32),
        pltpu.VMEM((NP,), _f32),
    ],
    compiler_params=pltpu.CompilerParams(needs_layout_passes=False),
)
def _deg_kernel(dst_hbm, out_hbm, idx_v, deg_v):
    c = lax.axis_index("c")
    s = lax.axis_index("s")
    wid = c * NS + s
    pltpu.sync_copy(dst_hbm.at[wid], idx_v)

    def zero(i, _):
        deg_v[pl.ds(i * 16, 16)] = jnp.zeros((16,), _f32)
        return 0

    lax.fori_loop(0, NP // 16, zero, 0)
    ones = jnp.ones((16,), _f32)

    def body(i, _):
        ids = idx_v[pl.ds(i * 16, 16)]
        plsc.addupdate_scatter(deg_v, [ids], ones)
        return 0

    lax.fori_loop(0, EPW // 16, body, 0)
    pltpu.sync_copy(deg_v, out_hbm.at[wid])


def _make_scatter(dh, nb, nchb, nch_total, edgesplit):
    npass = nch_total // nchb
    G = nchb // nb

    @functools.partial(
        pl.kernel,
        out_type=jax.ShapeDtypeStruct((NC, NP, dh), _f32),
        mesh=_mesh(),
        scratch_types=[
            pltpu.VMEM((nchb, K), jnp.int32),
            pltpu.VMEM((nchb, K), jnp.int32),
        ]
        + [pltpu.VMEM((128, dh), _f32) for _ in range(nb)]
        + [pltpu.VMEM_SHARED((NP, dh), _f32)]
        + [pltpu.SemaphoreType.DMA for _ in range(2 * nb)],
        compiler_params=pltpu.CompilerParams(use_tc_tiling_on_sc=False),
    )
    def k(g_hbm, src_hbm, dst_hbm, out_hbm, src_v, dst_v, *rest):
        rawbufs = rest[:nb]
        acc = rest[nb]
        gsems = rest[nb + 1 : 2 * nb + 1]
        ssems = rest[2 * nb + 1 :]
        c = lax.axis_index("c")
        s = lax.axis_index("s")
        bufs = tuple(rb.at[pl.ds(0, K)] for rb in rawbufs)
        buf0 = rawbufs[0]
        if edgesplit:
            gtab = g_hbm
            coff = c * nch_total
        else:
            gtab = g_hbm.at[c]
            coff = 0
        src_idx = src_hbm.at[s]
        dst_idx = dst_hbm.at[s]

        def issue_g(j, b):
            pltpu.async_copy(gtab.at[src_v.at[j]], bufs[b], gsems[b])

        def wait_g(j, b):
            pltpu.make_async_copy(gtab.at[src_v.at[j]], bufs[b], gsems[b]).wait()

        def issue_s(j, b):
            pltpu.async_copy(bufs[b], acc.at[dst_v.at[j]], ssems[b], add=True)

        def wait_s(j, b):
            pltpu.make_async_copy(bufs[b], acc.at[dst_v.at[j]], ssems[b]).wait()

        base = s * RPT
        if edgesplit:
            @pl.when(c == 0)
            def _():
                pltpu.sync_copy(gtab.at[pl.ds(base, RPT)], acc.at[pl.ds(base, RPT)])

            @pl.when(c != 0)
            def _():
                def zrow(i, _):
                    for j in range(dh // 16):
                        buf0[i, pl.ds(j * 16, 16)] = jnp.zeros((16,), _f32)
                    return 0

                lax.fori_loop(0, 128, zrow, 0)
                for r in range(RPT // 128):
                    pltpu.sync_copy(buf0, acc.at[pl.ds(base + r * 128, 128)])
        else:
            pltpu.sync_copy(gtab.at[pl.ds(base, RPT)], acc.at[pl.ds(base, RPT)])
        plsc.subcore_barrier()

        for p in range(npass):
            pltpu.sync_copy(src_idx.at[pl.ds(coff + p * nchb, nchb)], src_v)
            pltpu.sync_copy(dst_idx.at[pl.ds(coff + p * nchb, nchb)], dst_v)
            for b in range(nb):
                issue_g(b, b)

            def body(gi, _):
                for b in range(nb):
                    j = nb * gi + b
                    wait_g(j, b)
                    issue_s(j, b)
                    wait_s(j, b)
                    issue_g(j + nb, b)
                return 0

            lax.fori_loop(0, G - 1, body, 0)
            for b in range(nb):
                j = nb * (G - 1) + b
                wait_g(j, b)
                issue_s(j, b)
                wait_s(j, b)
        plsc.subcore_barrier()
        pltpu.sync_copy(acc.at[pl.ds(base, RPT)], out_hbm.at[c].at[pl.ds(base, RPT)])

    return k


def _tc1_body(deg_ref, x_ref, wa_ref, wb_ref, g_ref, dinv_ref):
    deg = jnp.sum(deg_ref[...], axis=0)[:, None] + 1.0
    dinv = lax.rsqrt(deg)
    dinv_ref[...] = dinv
    xa = x_ref[...]
    g_ref[0] = dinv * jnp.dot(xa, wa_ref[...], preferred_element_type=_f32)
    g_ref[1] = dinv * jnp.dot(xa, wb_ref[...], preferred_element_type=_f32)


def _tc_mid_body(s_ref, dinv_ref, b_ref, wa_ref, wb_ref, o_ref):
    dinv = dinv_ref[...]
    t = jnp.concatenate([s_ref[0], s_ref[1]], axis=1)
    a = jnp.maximum(dinv * t + b_ref[...], 0.0)
    o_ref[0] = dinv * jnp.dot(a, wa_ref[...], preferred_element_type=_f32)
    o_ref[1] = dinv * jnp.dot(a, wb_ref[...], preferred_element_type=_f32)


def _tc4_body(s_ref, dinv_ref, b_ref, w_ref, o_ref):
    dinv = dinv_ref[...]
    t = jnp.concatenate([s_ref[0], s_ref[1]], axis=1)
    a = jnp.maximum(dinv * t + b_ref[...], 0.0)
    o_ref[...] = dinv * jnp.dot(a, w_ref[...], preferred_element_type=_f32)


def _tc5_body(s_ref, dinv_ref, b_ref, o_ref):
    dinv = dinv_ref[...]
    logits = dinv * (s_ref[0] + s_ref[1]) + b_ref[...]
    m = jnp.max(logits, axis=1, keepdims=True)
    lse = jnp.log(jnp.sum(jnp.exp(logits - m), axis=1, keepdims=True)) + m
    o_ref[...] = logits - lse


def _pad2(w, rows, cols):
    return jnp.pad(w, ((0, rows - w.shape[0]), (0, cols - w.shape[1])))


def kernel(x, edge_index, W1, b1, W2, b2, W3, b3, W4, b4):
    x = jnp.pad(x, ((0, NP - N), (0, 0)))
    src = edge_index[0]
    dst = edge_index[1]
    dst_deg = dst.reshape(NW, EPW)
    src_t = src.reshape(NS, NCH, K)
    dst_t = dst.reshape(NS, NCH, K)

    w1 = _pad2(W1, 128, 224)
    w2 = _pad2(W2, 224, 160)
    w3 = _pad2(W3, 160, 64)
    w4 = _pad2(W4, 64, 16)
    b1p = jnp.pad(b1, (0, 4)).reshape(1, 224)
    b2p = jnp.pad(b2, (0, 10)).reshape(1, 160)
    b3p = jnp.pad(b3, (0, 4)).reshape(1, 64)
    b4p = b4.reshape(1, 16)

    deg_parts = _deg_kernel(dst_deg)

    g1, dinv = pl.pallas_call(
        _tc1_body,
        grid=(GRID,),
        in_specs=[
            pl.BlockSpec((NW, RB), lambda i: (0, i)),
            pl.BlockSpec((RB, 128), lambda i: (i, 0)),
            pl.BlockSpec((128, 112), lambda i: (0, 0)),
            pl.BlockSpec((128, 112), lambda i: (0, 0)),
        ],
        out_specs=[
            pl.BlockSpec((2, RB, 112), lambda i: (0, i, 0)),
            pl.BlockSpec((RB, 1), lambda i: (i, 0)),
        ],
        out_shape=[
            jax.ShapeDtypeStruct((2, NP, 112), _f32),
            jax.ShapeDtypeStruct((NP, 1), _f32),
        ],
    )(deg_parts, x, w1[:, :112], w1[:, 112:])

    s1 = _scat112(g1, src_t, dst_t)

    g2 = pl.pallas_call(
        _tc_mid_body,
        grid=(GRID,),
        in_specs=[
            pl.BlockSpec((2, RB, 112), lambda i: (0, i, 0)),
            pl.BlockSpec((RB, 1), lambda i: (i, 0)),
            pl.BlockSpec((1, 224), lambda i: (0, 0)),
            pl.BlockSpec((224, 80), lambda i: (0, 0)),
            pl.BlockSpec((224, 80), lambda i: (0, 0)),
        ],
        out_specs=pl.BlockSpec((2, RB, 80), lambda i: (0, i, 0)),
        out_shape=jax.ShapeDtypeStruct((2, NP, 80), _f32),
    )(s1, dinv, b1p, w2[:, :80], w2[:, 80:])

    s2 = _scat80(g2, src_t, dst_t)

    g3 = pl.pallas_call(
        _tc_mid_body,
        grid=(GRID,),
        in_specs=[
            pl.BlockSpec((2, RB, 80), lambda i: (0, i, 0)),
            pl.BlockSpec((RB, 1), lambda i: (i, 0)),
            pl.BlockSpec((1, 160), lambda i: (0, 0)),
            pl.BlockSpec((160, 32), lambda i: (0, 0)),
            pl.BlockSpec((160, 32), lambda i: (0, 0)),
        ],
        out_specs=pl.BlockSpec((2, RB, 32), lambda i: (0, i, 0)),
        out_shape=jax.ShapeDtypeStruct((2, NP, 32), _f32),
    )(s2, dinv, b2p, w3[:, :32], w3[:, 32:])

    s3 = _scat32(g3, src_t, dst_t)

    g4 = pl.pallas_call(
        _tc4_body,
        grid=(GRID,),
        in_specs=[
            pl.BlockSpec((2, RB, 32), lambda i: (0, i, 0)),
            pl.BlockSpec((RB, 1), lambda i: (i, 0)),
            pl.BlockSpec((1, 64), lambda i: (0, 0)),
            pl.BlockSpec((64, 16), lambda i: (0, 0)),
        ],
        out_specs=pl.BlockSpec((RB, 16), lambda i: (i, 0)),
        out_shape=jax.ShapeDtypeStruct((NP, 16), _f32),
    )(s3, dinv, b3p, w4)

    s4 = _scat16e(g4, src_t, dst_t)

    out = pl.pallas_call(
        _tc5_body,
        grid=(N // RB5,),
        in_specs=[
            pl.BlockSpec((2, RB5, 16), lambda i: (0, i, 0)),
            pl.BlockSpec((RB5, 1), lambda i: (i, 0)),
            pl.BlockSpec((1, 16), lambda i: (0, 0)),
        ],
        out_specs=pl.BlockSpec((RB5, 16), lambda i: (i, 0)),
        out_shape=jax.ShapeDtypeStruct((N, 16), _f32),
    )(s4, dinv, b4p)

    return out


_scat112 = _make_scatter(112, 2, 80, NCH, False)
_scat80 = _make_scatter(80, 4, 80, NCH, False)
_scat32 = _make_scatter(32, 4, 160, NCH, False)
_scat16e = _make_scatter(16, 4, 80, NCH4, True)


# speedup vs baseline: 2.0104x; 1.0232x over previous
import functools

import jax
import jax.numpy as jnp
from jax import lax
from jax.experimental import pallas as pl
from jax.experimental.pallas import tpu as pltpu
from jax.experimental.pallas import tpu_sc as plsc

N = 10000
NP = 10240
E = 320000
NC = 2
NS = 16
NW = NC * NS

K = 125
EPT = E // NS
NCH = EPT // K
NCHB = 80
EPW = E // NW
NCH4 = EPW // K
RPT = NP // NS
RB = 2048
GRID = NP // RB
RB5 = 2000

_f32 = jnp.float32


def _mesh():
    return plsc.VectorSubcoreMesh(
        core_axis_name="c", subcore_axis_name="s", num_cores=NC, num_subcores=NS
    )


@functools.partial(
    pl.kernel,
    out_type=jax.ShapeDtypeStruct((NW, NP), _f32),
    mesh=_mesh(),
    scratch_types=[
        pltpu.VMEM((EPW,), jnp.int32),
        pltpu.VMEM((NP,), _f32),
    ],
    compiler_params=pltpu.CompilerParams(needs_layout_passes=False),
)
def _deg_kernel(dst_hbm, out_hbm, idx_v, deg_v):
    c = lax.axis_index("c")
    s = lax.axis_index("s")
    wid = c * NS + s
    pltpu.sync_copy(dst_hbm.at[wid], idx_v)

    def zero(i, _):
        deg_v[pl.ds(i * 16, 16)] = jnp.zeros((16,), _f32)
        return 0

    lax.fori_loop(0, NP // 16, zero, 0)
    ones = jnp.ones((16,), _f32)

    def body(i, _):
        ids = idx_v[pl.ds(i * 16, 16)]
        plsc.addupdate_scatter(deg_v, [ids], ones)
        return 0

    lax.fori_loop(0, EPW // 16, body, 0)
    pltpu.sync_copy(deg_v, out_hbm.at[wid])


def _make_scatter(dh, nb, nchb, nch_total, edgesplit):
    npass = nch_total // nchb
    G = nchb // nb

    @functools.partial(
        pl.kernel,
        out_type=jax.ShapeDtypeStruct((NC, NP, dh), _f32),
        mesh=_mesh(),
        scratch_types=[
            pltpu.VMEM((nchb, K), jnp.int32),
            pltpu.VMEM((nchb, K), jnp.int32),
        ]
        + [pltpu.VMEM((128, dh), _f32) for _ in range(nb)]
        + [pltpu.VMEM_SHARED((NP, dh), _f32)]
        + [pltpu.SemaphoreType.DMA for _ in range(2 * nb)],
        compiler_params=pltpu.CompilerParams(use_tc_tiling_on_sc=False),
    )
    def k(g_hbm, src_hbm, dst_hbm, out_hbm, src_v, dst_v, *rest):
        rawbufs = rest[:nb]
        acc = rest[nb]
        gsems = rest[nb + 1 : 2 * nb + 1]
        ssems = rest[2 * nb + 1 :]
        c = lax.axis_index("c")
        s = lax.axis_index("s")
        bufs = tuple(rb.at[pl.ds(0, K)] for rb in rawbufs)
        buf0 = rawbufs[0]
        if edgesplit:
            gtab = g_hbm
            coff = c * nch_total
        else:
            gtab = g_hbm.at[c]
            coff = 0
        src_idx = src_hbm.at[s]
        dst_idx = dst_hbm.at[s]

        def issue_g(j, b):
            pltpu.async_copy(gtab.at[src_v.at[j]], bufs[b], gsems[b])

        def wait_g(j, b):
            pltpu.make_async_copy(gtab.at[src_v.at[j]], bufs[b], gsems[b]).wait()

        def issue_s(j, b):
            pltpu.async_copy(bufs[b], acc.at[dst_v.at[j]], ssems[b], add=True)

        def wait_s(j, b):
            pltpu.make_async_copy(bufs[b], acc.at[dst_v.at[j]], ssems[b]).wait()

        base = s * RPT
        if edgesplit:
            @pl.when(c == 0)
            def _():
                pltpu.sync_copy(gtab.at[pl.ds(base, RPT)], acc.at[pl.ds(base, RPT)])

            @pl.when(c != 0)
            def _():
                def zrow(i, _):
                    for j in range(dh // 16):
                        buf0[i, pl.ds(j * 16, 16)] = jnp.zeros((16,), _f32)
                    return 0

                lax.fori_loop(0, 128, zrow, 0)
                for r in range(RPT // 128):
                    pltpu.sync_copy(buf0, acc.at[pl.ds(base + r * 128, 128)])
        else:
            pltpu.sync_copy(gtab.at[pl.ds(base, RPT)], acc.at[pl.ds(base, RPT)])
        plsc.subcore_barrier()

        for p in range(npass):
            pltpu.sync_copy(src_idx.at[pl.ds(coff + p * nchb, nchb)], src_v)
            pltpu.sync_copy(dst_idx.at[pl.ds(coff + p * nchb, nchb)], dst_v)
            for b in range(nb):
                issue_g(b, b)

            def body(gi, _):
                for b in range(nb):
                    j = nb * gi + b
                    wait_g(j, b)
                    issue_s(j, b)
                    wait_s(j, b)
                    issue_g(j + nb, b)
                return 0

            lax.fori_loop(0, G - 1, body, 0)
            for b in range(nb):
                j = nb * (G - 1) + b
                wait_g(j, b)
                issue_s(j, b)
                wait_s(j, b)
        plsc.subcore_barrier()
        pltpu.sync_copy(acc.at[pl.ds(base, RPT)], out_hbm.at[c].at[pl.ds(base, RPT)])

    return k


def _tc1_body(deg_ref, x_ref, wa_ref, wb_ref, g_ref, dinv_ref):
    deg = jnp.sum(deg_ref[...], axis=0)[:, None] + 1.0
    dinv = lax.rsqrt(deg)
    dinv_ref[...] = dinv
    xa = x_ref[...]
    g_ref[0] = dinv * jnp.dot(xa, wa_ref[...], preferred_element_type=_f32)
    g_ref[1] = dinv * jnp.dot(xa, wb_ref[...], preferred_element_type=_f32)


def _tc_mid_body(s_ref, dinv_ref, b_ref, wa_ref, wb_ref, o_ref):
    dinv = dinv_ref[...]
    t = jnp.concatenate([s_ref[0], s_ref[1]], axis=1)
    a = jnp.maximum(dinv * t + b_ref[...], 0.0)
    o_ref[0] = dinv * jnp.dot(a, wa_ref[...], preferred_element_type=_f32)
    o_ref[1] = dinv * jnp.dot(a, wb_ref[...], preferred_element_type=_f32)


def _tc4_body(s_ref, dinv_ref, b_ref, w_ref, o_ref):
    dinv = dinv_ref[...]
    t = jnp.concatenate([s_ref[0], s_ref[1]], axis=1)
    a = jnp.maximum(dinv * t + b_ref[...], 0.0)
    o_ref[...] = dinv * jnp.dot(a, w_ref[...], preferred_element_type=_f32)


def _tc5_body(s_ref, dinv_ref, b_ref, o_ref):
    dinv = dinv_ref[...]
    logits = dinv * (s_ref[0] + s_ref[1]) + b_ref[...]
    m = jnp.max(logits, axis=1, keepdims=True)
    lse = jnp.log(jnp.sum(jnp.exp(logits - m), axis=1, keepdims=True)) + m
    o_ref[...] = logits - lse


def _pad2(w, rows, cols):
    return jnp.pad(w, ((0, rows - w.shape[0]), (0, cols - w.shape[1])))


def kernel(x, edge_index, W1, b1, W2, b2, W3, b3, W4, b4):
    x = jnp.pad(x, ((0, NP - N), (0, 0)))
    src = edge_index[0]
    dst = edge_index[1]
    dst_deg = dst.reshape(NW, EPW)
    src_t = src.reshape(NS, NCH, K)
    dst_t = dst.reshape(NS, NCH, K)

    w1 = _pad2(W1, 128, 224)
    w2 = _pad2(W2, 224, 160)
    w3 = _pad2(W3, 160, 64)
    w4 = _pad2(W4, 64, 16)
    b1p = jnp.pad(b1, (0, 4)).reshape(1, 224)
    b2p = jnp.pad(b2, (0, 10)).reshape(1, 160)
    b3p = jnp.pad(b3, (0, 4)).reshape(1, 64)
    b4p = b4.reshape(1, 16)

    deg_parts = _deg_kernel(dst_deg)

    g1, dinv = pl.pallas_call(
        _tc1_body,
        grid=(GRID,),
        in_specs=[
            pl.BlockSpec((NW, RB), lambda i: (0, i)),
            pl.BlockSpec((RB, 128), lambda i: (i, 0)),
            pl.BlockSpec((128, 112), lambda i: (0, 0)),
            pl.BlockSpec((128, 112), lambda i: (0, 0)),
        ],
        out_specs=[
            pl.BlockSpec((2, RB, 112), lambda i: (0, i, 0)),
            pl.BlockSpec((RB, 1), lambda i: (i, 0)),
        ],
        out_shape=[
            jax.ShapeDtypeStruct((2, NP, 112), _f32),
            jax.ShapeDtypeStruct((NP, 1), _f32),
        ],
    )(deg_parts, x, w1[:, :112], w1[:, 112:])

    s1 = _scat112(g1, src_t, dst_t)

    g2 = pl.pallas_call(
        _tc_mid_body,
        grid=(GRID,),
        in_specs=[
            pl.BlockSpec((2, RB, 112), lambda i: (0, i, 0)),
            pl.BlockSpec((RB, 1), lambda i: (i, 0)),
            pl.BlockSpec((1, 224), lambda i: (0, 0)),
            pl.BlockSpec((224, 80), lambda i: (0, 0)),
            pl.BlockSpec((224, 80), lambda i: (0, 0)),
        ],
        out_specs=pl.BlockSpec((2, RB, 80), lambda i: (0, i, 0)),
        out_shape=jax.ShapeDtypeStruct((2, NP, 80), _f32),
    )(s1, dinv, b1p, w2[:, :80], w2[:, 80:])

    s2 = _scat80(g2, src_t, dst_t)

    g3 = pl.pallas_call(
        _tc_mid_body,
        grid=(GRID,),
        in_specs=[
            pl.BlockSpec((2, RB, 80), lambda i: (0, i, 0)),
            pl.BlockSpec((RB, 1), lambda i: (i, 0)),
            pl.BlockSpec((1, 160), lambda i: (0, 0)),
            pl.BlockSpec((160, 32), lambda i: (0, 0)),
            pl.BlockSpec((160, 32), lambda i: (0, 0)),
        ],
        out_specs=pl.BlockSpec((2, RB, 32), lambda i: (0, i, 0)),
        out_shape=jax.ShapeDtypeStruct((2, NP, 32), _f32),
    )(s2, dinv, b2p, w3[:, :32], w3[:, 32:])

    s3 = _scat32(g3, src_t, dst_t)

    g4 = pl.pallas_call(
        _tc4_body,
        grid=(GRID,),
        in_specs=[
            pl.BlockSpec((2, RB, 32), lambda i: (0, i, 0)),
            pl.BlockSpec((RB, 1), lambda i: (i, 0)),
            pl.BlockSpec((1, 64), lambda i: (0, 0)),
            pl.BlockSpec((64, 16), lambda i: (0, 0)),
        ],
        out_specs=pl.BlockSpec((RB, 16), lambda i: (i, 0)),
        out_shape=jax.ShapeDtypeStruct((NP, 16), _f32),
    )(s3, dinv, b3p, w4)

    s4 = _scat16e(g4, src_t, dst_t)

    out = pl.pallas_call(
        _tc5_body,
        grid=(N // RB5,),
        in_specs=[
            pl.BlockSpec((2, RB5, 16), lambda i: (0, i, 0)),
            pl.BlockSpec((RB5, 1), lambda i: (i, 0)),
            pl.BlockSpec((1, 16), lambda i: (0, 0)),
        ],
        out_specs=pl.BlockSpec((RB5, 16), lambda i: (i, 0)),
        out_shape=jax.ShapeDtypeStruct((N, 16), _f32),
    )(s4, dinv, b4p)

    return out


_scat112 = _make_scatter(112, 2, 80, NCH, False)
_scat80 = _make_scatter(80, 4, 80, NCH, False)
_scat32 = _make_scatter(32, 8, 160, NCH, False)
_scat16e = _make_scatter(16, 8, 80, NCH4, True)
